# baseline (speedup 1.0000x reference)
import functools

import jax
import jax.numpy as jnp
from jax import lax
from jax.experimental import pallas as pl
from jax.experimental.pallas import tpu as pltpu
from jax.experimental.pallas import tpu_sc as plsc

_C = 256
_CW = _C // 2
_K = 16
_L = 16
_NC, _NS = 2, 16
_NW = _NC * _NS
_CH = 4
_CHK = _CH * _K
_NBUF = 2
_HI = -65536


def _f32(w):
    return lax.bitcast_convert_type(w, jnp.float32)


def _sc_max_relative(xsT_w, e0, e1, n_pad):
    rows_per_w = n_pad // _NW
    n_chunks = rows_per_w // _CH
    assert n_chunks % _NBUF == 0
    mesh = plsc.VectorSubcoreMesh(core_axis_name="c", subcore_axis_name="s",
                                  num_cores=_NC)

    @functools.partial(
        pl.kernel,
        out_type=(jax.ShapeDtypeStruct((n_pad // 2, 2, _CW), jnp.float32),
                  jax.ShapeDtypeStruct((n_pad // 2, 2, _CW), jnp.float32)),
        mesh=mesh,
        scratch_types=[
            pltpu.VMEM_SHARED((10000, _CW), jnp.int32),
            pltpu.VMEM((rows_per_w * _K,), jnp.int32),
            pltpu.VMEM((rows_per_w * _K,), jnp.int32),
            [pltpu.VMEM((_CHK, _CW), jnp.int32)] * _NBUF,
            [pltpu.VMEM((_CHK, _CW), jnp.int32)] * _NBUF,
            [pltpu.VMEM((_CH, 2, _CW), jnp.float32)] * _NBUF,
            [pltpu.SemaphoreType.DMA] * _NBUF,
            [pltpu.SemaphoreType.DMA] * _NBUF,
            [pltpu.SemaphoreType.DMA] * _NBUF,
        ],
    )
    def sc_body(xsT_hbm, e0_hbm, e1_hbm, out_a_hbm, out_b_hbm,
                tbl, idx0, idx1, srcs, dsts, mbufs, ssems, dsems, osems):
        cid = lax.axis_index("c")
        sid = lax.axis_index("s")
        wid = sid * _NC + cid
        base_node = wid * rows_per_w
        base_local = sid * rows_per_w

        @pl.when(lax.axis_index("s") == 0)
        def _():
            pltpu.sync_copy(xsT_hbm, tbl)

        pltpu.sync_copy(e0_hbm.at[pl.ds(base_node * _K, rows_per_w * _K)], idx0)
        pltpu.sync_copy(e1_hbm.at[pl.ds(base_node * _K, rows_per_w * _K)], idx1)
        plsc.subcore_barrier()

        def gather(t, p):
            tm = lax.rem(t, n_chunks)
            s = pl.ds(tm * _CHK, _CHK)
            pltpu.async_copy(tbl.at[idx0.at[s]], srcs[p], ssems[p])
            pltpu.async_copy(tbl.at[idx1.at[s]], dsts[p], dsems[p])

        def gather_wait(p):
            pltpu.make_async_copy(tbl.at[idx0.at[pl.ds(0, _CHK)]],
                                  srcs[p], ssems[p]).wait()
            pltpu.make_async_copy(tbl.at[idx1.at[pl.ds(0, _CHK)]],
                                  dsts[p], dsems[p]).wait()

        def wb_wait(p):
            pltpu.make_async_copy(mbufs[p], out_a_hbm.at[pl.ds(0, _CH)],
                                  osems[p]).wait()

        for p in range(_NBUF):
            gather(p, p)

        def half(t, p):
            src, dst, mbuf = srcs[p], dsts[p], mbufs[p]
            gather_wait(p)

            def node(i, carry):
                r = i * _K
                for g in range(_CW // _L):
                    sl = pl.ds(g * _L, _L)
                    wl = src[r, sl]
                    wr = dst[r, sl]
                    acc_e = _f32(wl << 16) - _f32(wr << 16)
                    acc_o = _f32(wl & _HI) - _f32(wr & _HI)
                    for k in range(1, _K):
                        wl = src[r + k, sl]
                        wr = dst[r + k, sl]
                        acc_e = jnp.maximum(acc_e, _f32(wl << 16) - _f32(wr << 16))
                        acc_o = jnp.maximum(acc_o, _f32(wl & _HI) - _f32(wr & _HI))
                    mbuf[i, 0, sl] = acc_e
                    mbuf[i, 1, sl] = acc_o
                return carry

            lax.fori_loop(0, _CH, node, 0)
            gather(t + _NBUF, p)
            nb = base_local + t * _CH

            @pl.when(cid == 0)
            def _():
                pltpu.async_copy(mbuf, out_a_hbm.at[pl.ds(nb, _CH)], osems[p])

            @pl.when(cid == 1)
            def _():
                pltpu.async_copy(mbuf, out_b_hbm.at[pl.ds(nb, _CH)], osems[p])

        def ring(q, carry):
            t = q * _NBUF
            @pl.when(q > 0)
            def _():
                for p in range(_NBUF):
                    wb_wait(p)
            for p in range(_NBUF):
                half(t + p, p)
            return carry

        lax.fori_loop(0, n_chunks // _NBUF, ring, 0)

        for p in range(_NBUF):
            gather_wait(p)
            wb_wait(p)

    return sc_body(xsT_w, e0, e1)


def _tc_conv(xs, m_pad, We, Wo_re, b2, n):
    c_out = We.shape[0]
    nb = 2048
    grid = (pl.cdiv(n, nb),)

    def body(xs_ref, m_ref, we_ref, wo_ref, b_ref, o_ref):
        acc = lax.dot_general(we_ref[...], xs_ref[...],
                              (((1,), (0,)), ((), ())),
                              preferred_element_type=jnp.float32)
        acc = acc + lax.dot_general(wo_ref[...], m_ref[...],
                                    (((1,), (1,)), ((), ())),
                                    preferred_element_type=jnp.float32)
        o_ref[...] = jnp.maximum(acc + b_ref[...], 0.0)

    return pl.pallas_call(
        body,
        grid=grid,
        in_specs=[
            pl.BlockSpec((_C, nb), lambda i: (0, i)),
            pl.BlockSpec((nb, _C), lambda i: (i, 0)),
            pl.BlockSpec((c_out, _C), lambda i: (0, 0)),
            pl.BlockSpec((c_out, _C), lambda i: (0, 0)),
            pl.BlockSpec((c_out, 1), lambda i: (0, 0)),
        ],
        out_specs=pl.BlockSpec((c_out, nb), lambda i: (0, i)),
        out_shape=jax.ShapeDtypeStruct((c_out, n), jnp.float32),
    )(xs, m_pad, We, Wo_re, b2)


def kernel(x, edge_index, W, b):
    B, C, N, _ = x.shape
    xs = x.reshape(C, N)
    xsT_bf = xs.T.astype(jnp.bfloat16)
    xsT_w = lax.bitcast_convert_type(
        xsT_bf.reshape(N, C // 2, 2), jnp.int32)
    n_pad = ((N + 8 * _NW * _CH - 1) // (8 * _NW * _CH)) * (8 * _NW * _CH)
    pad_e = n_pad * _K - N * _K
    e0 = jnp.pad(edge_index[0].reshape(N * _K), (0, pad_e))
    e1 = jnp.pad(edge_index[1].reshape(N * _K), (0, pad_e))

    m_a, m_b = _sc_max_relative(xsT_w, e0, e1, n_pad)
    half = n_pad // 2
    m_w = jnp.stack([m_a.reshape(_NS, half // _NS, 2, _CW),
                     m_b.reshape(_NS, half // _NS, 2, _CW)],
                    axis=1)
    m_pad = m_w.reshape(n_pad, C)

    We = W[:, 0::2]
    Wo = W[:, 1::2]
    Wo_re = jnp.concatenate([Wo[:, 0::2], Wo[:, 1::2]], axis=1)
    out = _tc_conv(xs, m_pad, We, Wo_re, b.reshape(-1, 1), N)
    return out.reshape(B, W.shape[0], N, 1)


# speedup vs baseline: 1.1587x; 1.1587x over previous
import functools

import jax
import jax.numpy as jnp
from jax import lax
from jax.experimental import pallas as pl
from jax.experimental.pallas import tpu as pltpu
from jax.experimental.pallas import tpu_sc as plsc

_C = 256
_CW = _C // 2
_K = 16
_L = 16
_NC, _NS = 2, 16
_NW = _NC * _NS
_CH = 4
_CHK = _CH * _K
_NBUF = 2
_HI = -65536


def _f32(w):
    return lax.bitcast_convert_type(w, jnp.float32)


def _sc_max_relative(xsT_w, e0, e1, n_pad):
    rows_per_w = n_pad // _NW
    n_chunks = rows_per_w // _CH
    assert n_chunks % _NBUF == 0
    mesh = plsc.VectorSubcoreMesh(core_axis_name="c", subcore_axis_name="s",
                                  num_cores=_NC)

    @functools.partial(
        pl.kernel,
        out_type=jax.ShapeDtypeStruct((n_pad, 2, _CW), jnp.float32),
        mesh=mesh,
        scratch_types=[
            pltpu.VMEM_SHARED((10000, _CW), jnp.int32),
            pltpu.VMEM((rows_per_w * _K,), jnp.int32),
            pltpu.VMEM((rows_per_w * _K,), jnp.int32),
            [pltpu.VMEM((_CHK, _CW), jnp.int32)] * _NBUF,
            [pltpu.VMEM((_CHK, _CW), jnp.int32)] * _NBUF,
            [pltpu.VMEM((_CH, 2, _CW), jnp.float32)] * _NBUF,
            [pltpu.SemaphoreType.DMA] * _NBUF,
            [pltpu.SemaphoreType.DMA] * _NBUF,
            [pltpu.SemaphoreType.DMA] * _NBUF,
        ],
    )
    def sc_body(xsT_hbm, e0_hbm, e1_hbm, out_hbm,
                tbl, idx0, idx1, srcs, dsts, mbufs, ssems, dsems, osems):
        wid = lax.axis_index("s") * _NC + lax.axis_index("c")
        base_node = wid * rows_per_w

        @pl.when(lax.axis_index("s") == 0)
        def _():
            pltpu.sync_copy(xsT_hbm, tbl)

        pltpu.sync_copy(e0_hbm.at[pl.ds(base_node * _K, rows_per_w * _K)], idx0)
        pltpu.sync_copy(e1_hbm.at[pl.ds(base_node * _K, rows_per_w * _K)], idx1)
        plsc.subcore_barrier()

        def gather(t, p):
            tm = lax.rem(t, n_chunks)
            s = pl.ds(tm * _CHK, _CHK)
            pltpu.async_copy(tbl.at[idx0.at[s]], srcs[p], ssems[p])
            pltpu.async_copy(tbl.at[idx1.at[s]], dsts[p], dsems[p])

        def gather_wait(p):
            pltpu.make_async_copy(tbl.at[idx0.at[pl.ds(0, _CHK)]],
                                  srcs[p], ssems[p]).wait()
            pltpu.make_async_copy(tbl.at[idx1.at[pl.ds(0, _CHK)]],
                                  dsts[p], dsems[p]).wait()

        def wb_wait(p):
            pltpu.make_async_copy(mbufs[p], out_hbm.at[pl.ds(0, _CH)],
                                  osems[p]).wait()

        for p in range(_NBUF):
            gather(p, p)

        def half(t, p):
            src, dst, mbuf = srcs[p], dsts[p], mbufs[p]
            gather_wait(p)

            def node(i, carry):
                r = i * _K
                for g in range(_CW // _L):
                    sl = pl.ds(g * _L, _L)
                    wl = src[r, sl]
                    wr = dst[r, sl]
                    acc_e = _f32(wl << 16) - _f32(wr << 16)
                    acc_o = _f32(wl & _HI) - _f32(wr & _HI)
                    for k in range(1, _K):
                        wl = src[r + k, sl]
                        wr = dst[r + k, sl]
                        acc_e = jnp.maximum(acc_e, _f32(wl << 16) - _f32(wr << 16))
                        acc_o = jnp.maximum(acc_o, _f32(wl & _HI) - _f32(wr & _HI))
                    mbuf[i, 0, sl] = acc_e
                    mbuf[i, 1, sl] = acc_o
                return carry

            lax.fori_loop(0, _CH, node, 0)
            gather(t + _NBUF, p)
            nb = base_node + t * _CH
            pltpu.async_copy(mbuf, out_hbm.at[pl.ds(nb, _CH)], osems[p])

        def ring(q, carry):
            t = q * _NBUF
            @pl.when(q > 0)
            def _():
                for p in range(_NBUF):
                    wb_wait(p)
            for p in range(_NBUF):
                half(t + p, p)
            return carry

        lax.fori_loop(0, n_chunks // _NBUF, ring, 0)

        for p in range(_NBUF):
            gather_wait(p)
            wb_wait(p)

    return sc_body(xsT_w, e0, e1)


def _tc_pad_edges(edge_index, n_pad):
    _, B, n, k = edge_index.shape
    e2 = edge_index.reshape(2, n, k)
    nb = 1024
    grid = (2, pl.cdiv(n_pad, nb))

    def body(e_ref, o_ref):
        j = pl.program_id(1)
        rows = jax.lax.broadcasted_iota(jnp.int32, (nb, k), 0) + j * nb
        o_ref[...] = jnp.where(rows < n, e_ref[...], 0)

    out = pl.pallas_call(
        body,
        grid=grid,
        in_specs=[pl.BlockSpec((1, nb, k), lambda i, j: (i, j, 0))],
        out_specs=pl.BlockSpec((1, nb, k), lambda i, j: (i, j, 0)),
        out_shape=jax.ShapeDtypeStruct((2, n_pad, k), jnp.int32),
    )(e2.reshape(2, n, k))
    return out.reshape(2, n_pad * k)


def _tc_pack_table(x2, n):
    nb = 1024
    grid = (pl.cdiv(n, nb),)

    def body(x_ref, o_ref):
        a = x_ref[:, 0, :].astype(jnp.bfloat16)
        bch = x_ref[:, 1, :].astype(jnp.bfloat16)
        ia = lax.convert_element_type(
            lax.bitcast_convert_type(a, jnp.uint16), jnp.uint32)
        ib = lax.convert_element_type(
            lax.bitcast_convert_type(bch, jnp.uint16), jnp.uint32)
        w = lax.bitcast_convert_type((ib << 16) | ia, jnp.int32)
        o_ref[...] = w.T

    return pl.pallas_call(
        body,
        grid=grid,
        in_specs=[pl.BlockSpec((_CW, 2, nb), lambda i: (0, 0, i))],
        out_specs=pl.BlockSpec((nb, _CW), lambda i: (i, 0)),
        out_shape=jax.ShapeDtypeStruct((n, _CW), jnp.int32),
    )(x2)


def _tc_conv(xs, m_pad, We, Wo_re, b2, n):
    c_out = We.shape[0]
    nb = 2048
    grid = (pl.cdiv(n, nb),)

    def body(xs_ref, m_ref, we_ref, wo_ref, b_ref, o_ref):
        acc = lax.dot_general(we_ref[...], xs_ref[...],
                              (((1,), (0,)), ((), ())),
                              preferred_element_type=jnp.float32)
        acc = acc + lax.dot_general(wo_ref[...], m_ref[...],
                                    (((1,), (1,)), ((), ())),
                                    preferred_element_type=jnp.float32)
        o_ref[...] = jnp.maximum(acc + b_ref[...], 0.0)

    return pl.pallas_call(
        body,
        grid=grid,
        in_specs=[
            pl.BlockSpec((_C, nb), lambda i: (0, i)),
            pl.BlockSpec((nb, _C), lambda i: (i, 0)),
            pl.BlockSpec((c_out, _C), lambda i: (0, 0)),
            pl.BlockSpec((c_out, _C), lambda i: (0, 0)),
            pl.BlockSpec((c_out, 1), lambda i: (0, 0)),
        ],
        out_specs=pl.BlockSpec((c_out, nb), lambda i: (0, i)),
        out_shape=jax.ShapeDtypeStruct((c_out, n), jnp.float32),
    )(xs, m_pad, We, Wo_re, b2)


def kernel(x, edge_index, W, b):
    B, C, N, _ = x.shape
    xs = x.reshape(C, N)
    n_pad = ((N + 8 * _NW * _CH - 1) // (8 * _NW * _CH)) * (8 * _NW * _CH)
    xsT_w = _tc_pack_table(x.reshape(C // 2, 2, N), N)
    e_pad = _tc_pad_edges(edge_index, n_pad)
    e0 = e_pad[0]
    e1 = e_pad[1]

    m_w = _sc_max_relative(xsT_w, e0, e1, n_pad)
    m_pad = m_w.reshape(n_pad, C)

    We = W[:, 0::2]
    Wo = W[:, 1::2]
    Wo_re = jnp.concatenate([Wo[:, 0::2], Wo[:, 1::2]], axis=1)
    out = _tc_conv(xs, m_pad, We, Wo_re, b.reshape(-1, 1), N)
    return out.reshape(B, W.shape[0], N, 1)


# speedup vs baseline: 1.1787x; 1.0173x over previous
import functools

import jax
import jax.numpy as jnp
from jax import lax
from jax.experimental import pallas as pl
from jax.experimental.pallas import tpu as pltpu
from jax.experimental.pallas import tpu_sc as plsc

_C = 256
_CW = _C // 2
_K = 16
_L = 16
_NC, _NS = 2, 16
_NW = _NC * _NS
_CH = 4
_CHK = _CH * _K
_NBUF = 2
_HI = -65536


def _f32(w):
    return lax.bitcast_convert_type(w, jnp.float32)


def _sc_max_relative(xsT_w, e_pad, n_pad):
    rows_per_w = n_pad // _NW
    n_chunks = rows_per_w // _CH
    assert n_chunks % _NBUF == 0
    mesh = plsc.VectorSubcoreMesh(core_axis_name="c", subcore_axis_name="s",
                                  num_cores=_NC)

    @functools.partial(
        pl.kernel,
        out_type=jax.ShapeDtypeStruct((n_pad, 2, _CW), jnp.float32),
        mesh=mesh,
        scratch_types=[
            pltpu.VMEM_SHARED((10000, _CW), jnp.int32),
            pltpu.VMEM((rows_per_w * _K,), jnp.int32),
            pltpu.VMEM((rows_per_w * _K,), jnp.int32),
            [pltpu.VMEM((_CHK, _CW), jnp.int32)] * _NBUF,
            [pltpu.VMEM((_CHK, _CW), jnp.int32)] * _NBUF,
            [pltpu.VMEM((_CH, 2, _CW), jnp.float32)] * _NBUF,
            [pltpu.SemaphoreType.DMA] * _NBUF,
            [pltpu.SemaphoreType.DMA] * _NBUF,
            [pltpu.SemaphoreType.DMA] * _NBUF,
        ],
    )
    def sc_body(xsT_hbm, e_hbm, out_hbm,
                tbl, idx0, idx1, srcs, dsts, mbufs, ssems, dsems, osems):
        wid = lax.axis_index("s") * _NC + lax.axis_index("c")
        base_node = wid * rows_per_w

        @pl.when(lax.axis_index("s") == 0)
        def _():
            pltpu.sync_copy(xsT_hbm, tbl)

        pltpu.sync_copy(e_hbm.at[0, pl.ds(base_node * _K, rows_per_w * _K)],
                        idx0)
        pltpu.sync_copy(e_hbm.at[1, pl.ds(base_node * _K, rows_per_w * _K)],
                        idx1)
        plsc.subcore_barrier()

        def gather(t, p):
            tm = lax.rem(t, n_chunks)
            s = pl.ds(tm * _CHK, _CHK)
            pltpu.async_copy(tbl.at[idx0.at[s]], srcs[p], ssems[p])
            pltpu.async_copy(tbl.at[idx1.at[s]], dsts[p], dsems[p])

        def gather_wait(p):
            pltpu.make_async_copy(tbl.at[idx0.at[pl.ds(0, _CHK)]],
                                  srcs[p], ssems[p]).wait()
            pltpu.make_async_copy(tbl.at[idx1.at[pl.ds(0, _CHK)]],
                                  dsts[p], dsems[p]).wait()

        def wb_wait(p):
            pltpu.make_async_copy(mbufs[p], out_hbm.at[pl.ds(0, _CH)],
                                  osems[p]).wait()

        for p in range(_NBUF):
            gather(p, p)

        def half(t, p):
            src, dst, mbuf = srcs[p], dsts[p], mbufs[p]
            gather_wait(p)

            def node(i, carry):
                r = i * _K
                for g in range(_CW // _L):
                    sl = pl.ds(g * _L, _L)
                    wl = src[r, sl]
                    wr = dst[r, sl]
                    acc_e = _f32(wl << 16) - _f32(wr << 16)
                    acc_o = _f32(wl & _HI) - _f32(wr & _HI)
                    for k in range(1, _K):
                        wl = src[r + k, sl]
                        wr = dst[r + k, sl]
                        acc_e = jnp.maximum(acc_e, _f32(wl << 16) - _f32(wr << 16))
                        acc_o = jnp.maximum(acc_o, _f32(wl & _HI) - _f32(wr & _HI))
                    mbuf[i, 0, sl] = acc_e
                    mbuf[i, 1, sl] = acc_o
                return carry

            lax.fori_loop(0, _CH, node, 0)
            gather(t + _NBUF, p)
            nb = base_node + t * _CH
            pltpu.async_copy(mbuf, out_hbm.at[pl.ds(nb, _CH)], osems[p])

        def ring(q, carry):
            t = q * _NBUF
            @pl.when(q > 0)
            def _():
                for p in range(_NBUF):
                    wb_wait(p)
            for p in range(_NBUF):
                half(t + p, p)
            return carry

        lax.fori_loop(0, n_chunks // _NBUF, ring, 0)

        for p in range(_NBUF):
            gather_wait(p)
            wb_wait(p)

    return sc_body(xsT_w, e_pad)


def _tc_pad_edges(edge_index, n_pad):
    _, B, n, k = edge_index.shape
    e2 = edge_index.reshape(2, n, k)
    nb = 1024
    grid = (2, pl.cdiv(n_pad, nb))

    def body(e_ref, o_ref):
        j = pl.program_id(1)
        rows = jax.lax.broadcasted_iota(jnp.int32, (nb, k), 0) + j * nb
        o_ref[...] = jnp.where(rows < n, e_ref[...], 0)

    out = pl.pallas_call(
        body,
        grid=grid,
        in_specs=[pl.BlockSpec((1, nb, k), lambda i, j: (i, j, 0))],
        out_specs=pl.BlockSpec((1, nb, k), lambda i, j: (i, j, 0)),
        out_shape=jax.ShapeDtypeStruct((2, n_pad, k), jnp.int32),
    )(e2.reshape(2, n, k))
    return out.reshape(2, n_pad * k)


def _tc_pack_table(x2, n):
    nb = 1024
    grid = (pl.cdiv(n, nb),)

    def body(x_ref, o_ref):
        a = x_ref[:, 0, :].astype(jnp.bfloat16)
        bch = x_ref[:, 1, :].astype(jnp.bfloat16)
        ia = lax.convert_element_type(
            lax.bitcast_convert_type(a, jnp.uint16), jnp.uint32)
        ib = lax.convert_element_type(
            lax.bitcast_convert_type(bch, jnp.uint16), jnp.uint32)
        w = lax.bitcast_convert_type((ib << 16) | ia, jnp.int32)
        o_ref[...] = w.T

    return pl.pallas_call(
        body,
        grid=grid,
        in_specs=[pl.BlockSpec((_CW, 2, nb), lambda i: (0, 0, i))],
        out_specs=pl.BlockSpec((nb, _CW), lambda i: (i, 0)),
        out_shape=jax.ShapeDtypeStruct((n, _CW), jnp.int32),
    )(x2)


def _tc_conv(xs, m_pad, We, Wo_re, b2, n):
    c_out = We.shape[0]
    nb = 2048
    grid = (pl.cdiv(n, nb),)

    def body(xs_ref, m_ref, we_ref, wo_ref, b_ref, o_ref):
        acc = lax.dot_general(we_ref[...], xs_ref[...],
                              (((1,), (0,)), ((), ())),
                              preferred_element_type=jnp.float32)
        acc = acc + lax.dot_general(wo_ref[...], m_ref[...],
                                    (((1,), (1,)), ((), ())),
                                    preferred_element_type=jnp.float32)
        o_ref[...] = jnp.maximum(acc + b_ref[...], 0.0)

    return pl.pallas_call(
        body,
        grid=grid,
        in_specs=[
            pl.BlockSpec((_C, nb), lambda i: (0, i)),
            pl.BlockSpec((nb, _C), lambda i: (i, 0)),
            pl.BlockSpec((c_out, _C), lambda i: (0, 0)),
            pl.BlockSpec((c_out, _C), lambda i: (0, 0)),
            pl.BlockSpec((c_out, 1), lambda i: (0, 0)),
        ],
        out_specs=pl.BlockSpec((c_out, nb), lambda i: (0, i)),
        out_shape=jax.ShapeDtypeStruct((c_out, n), jnp.float32),
    )(xs, m_pad, We, Wo_re, b2)


def kernel(x, edge_index, W, b):
    B, C, N, _ = x.shape
    xs = x.reshape(C, N)
    n_pad = ((N + 8 * _NW * _CH - 1) // (8 * _NW * _CH)) * (8 * _NW * _CH)
    xsT_w = _tc_pack_table(x.reshape(C // 2, 2, N), N)
    e_pad = _tc_pad_edges(edge_index, n_pad)

    m_w = _sc_max_relative(xsT_w, e_pad, n_pad)
    m_pad = m_w.reshape(n_pad, C)

    We = W[:, 0::2]
    Wo = W[:, 1::2]
    Wo_re = jnp.concatenate([Wo[:, 0::2], Wo[:, 1::2]], axis=1)
    out = _tc_conv(xs, m_pad, We, Wo_re, b.reshape(-1, 1), N)
    return out.reshape(B, W.shape[0], N, 1)
